# baseline (speedup 1.0000x reference)
import jax
import jax.numpy as jnp
from jax.experimental import pallas as pl

B, T, N, D = 16, 2048, 8, 128
TBLK = 1024


def _body(bp_ref, w_ref, out_ref):
    bp = bp_ref[...].astype(jnp.float32)
    out_ref[...] = bp[:, :, None] * w_ref[...][None, :, :]


def kernel(button_presses, W):
    bp = button_presses.reshape(B * T, N)
    out = pl.pallas_call(
        _body,
        grid=(B * T // TBLK,),
        in_specs=[
            pl.BlockSpec((TBLK, N), lambda i: (i, 0)),
            pl.BlockSpec((N, D), lambda i: (0, 0)),
        ],
        out_specs=pl.BlockSpec((TBLK, N, D), lambda i: (i, 0, 0)),
        out_shape=jax.ShapeDtypeStruct((B * T, N, D), jnp.float32),
    )(bp, W)
    return out.reshape(B, T, N, D)


# speedup vs baseline: 1.1369x; 1.1369x over previous
import jax
import jax.numpy as jnp
from jax.experimental import pallas as pl

B, T, N, D = 16, 2048, 8, 128
TBLK = 4096


def _body(bp_ref, w_ref, out_ref):
    bp = bp_ref[...].astype(jnp.float32)
    out_ref[...] = bp[:, :, None] * w_ref[...][None, :, :]


def kernel(button_presses, W):
    bp = button_presses.reshape(B * T, N)
    out = pl.pallas_call(
        _body,
        grid=(B * T // TBLK,),
        in_specs=[
            pl.BlockSpec((TBLK, N), lambda i: (i, 0)),
            pl.BlockSpec((N, D), lambda i: (0, 0)),
        ],
        out_specs=pl.BlockSpec((TBLK, N, D), lambda i: (i, 0, 0)),
        out_shape=jax.ShapeDtypeStruct((B * T, N, D), jnp.float32),
    )(bp, W)
    return out.reshape(B, T, N, D)
